# baseline (speedup 1.0000x reference)
import functools

import jax
import jax.numpy as jnp
from jax.experimental import pallas as pl
from jax.experimental.pallas import tpu as pltpu
from jax.experimental.pallas import tpu_sc as plsc

_B, _S, _D = 4096, 50, 64
_N = _B * _S
_V = 100001
_W = 400
_TT = 8192
_SB = 5


def _prep_body(i_ref, o_ref):
    t = jnp.transpose(i_ref[...], (1, 0))
    o_ref[...] = jnp.concatenate([t, t], axis=1)


def _swap_body(i_ref, o_ref):
    o_ref[...] = jnp.transpose(i_ref[...], (0, 2, 1))


def kernel(x, emb_weight):
    idx2 = (x.T.reshape(_N).astype(jnp.int32)) * 2

    table = pl.pallas_call(
        _prep_body,
        grid=(pl.cdiv(_V, _TT),),
        in_specs=[pl.BlockSpec((_D, _TT), lambda i: (0, i))],
        out_specs=pl.BlockSpec((_TT, 128), lambda i: (i, 0)),
        out_shape=jax.ShapeDtypeStruct((_V, 128), emb_weight.dtype),
    )(emb_weight.T).reshape(2 * _V, _D)

    @functools.partial(
        pl.kernel,
        out_type=jax.ShapeDtypeStruct((_N, _D), emb_weight.dtype),
        mesh=plsc.VectorSubcoreMesh(core_axis_name="c", subcore_axis_name="s"),
        compiler_params=pltpu.CompilerParams(use_tc_tiling_on_sc=False),
    )
    def gather_kernel(table_hbm, idx_hbm, out_hbm):
        def body(idx_vmem, out_vmem):
            pltpu.sync_copy(table_hbm.at[idx_vmem], out_vmem)

        pltpu.emit_pipeline(
            body,
            grid=(_N // _W,),
            in_specs=[pl.BlockSpec((_W,), index_map=lambda i: (i,))],
            out_specs=[pl.BlockSpec((_W, _D), index_map=lambda i: (i, 0))],
            core_axis_name=("c", "s"),
            dimension_semantics=(pltpu.PARALLEL,),
        )(idx_hbm, out_hbm)

    wide = gather_kernel(table, idx2).reshape(_S, _B, _D)

    swapped = pl.pallas_call(
        _swap_body,
        grid=(_S // _SB,),
        in_specs=[pl.BlockSpec((_SB, _B, _D), lambda i: (i, 0, 0))],
        out_specs=pl.BlockSpec((_SB, _D, _B), lambda i: (i, 0, 0)),
        out_shape=jax.ShapeDtypeStruct((_S, _D, _B), emb_weight.dtype),
    )(wide)

    return swapped.transpose(2, 0, 1)


# speedup vs baseline: 1.1244x; 1.1244x over previous
import functools

import jax
import jax.numpy as jnp
from jax.experimental import pallas as pl
from jax.experimental.pallas import tpu as pltpu
from jax.experimental.pallas import tpu_sc as plsc

_B, _S, _D = 4096, 50, 64
_N = _B * _S
_V = 100001
_H = _B // 2
_W = 400
_TT = 8192
_SB = 5


def _prep_body(i_ref, o_ref):
    t = jnp.transpose(i_ref[...], (1, 0))
    o_ref[...] = jnp.concatenate([t, t], axis=1)


def _swap_body(i_ref, o_ref):
    o_ref[:, :, :_H] = jnp.transpose(i_ref[:, :, : _D], (0, 2, 1))
    o_ref[:, :, _H:] = jnp.transpose(i_ref[:, :, _D:], (0, 2, 1))


def kernel(x, emb_weight):
    idx = x.T.reshape(_S, 2, _H).transpose(0, 2, 1).reshape(_N)
    idx = idx.astype(jnp.int32) * 2

    table = pl.pallas_call(
        _prep_body,
        grid=(pl.cdiv(_V, _TT),),
        in_specs=[pl.BlockSpec((_D, _TT), lambda i: (0, i))],
        out_specs=pl.BlockSpec((_TT, 128), lambda i: (i, 0)),
        out_shape=jax.ShapeDtypeStruct((_V, 128), emb_weight.dtype),
    )(emb_weight.T).reshape(2 * _V, _D)

    @functools.partial(
        pl.kernel,
        out_type=jax.ShapeDtypeStruct((_N, _D), emb_weight.dtype),
        mesh=plsc.VectorSubcoreMesh(core_axis_name="c", subcore_axis_name="s"),
        compiler_params=pltpu.CompilerParams(use_tc_tiling_on_sc=False),
    )
    def gather_kernel(table_hbm, idx_hbm, out_hbm):
        def body(idx_vmem, out_vmem):
            pltpu.sync_copy(table_hbm.at[idx_vmem], out_vmem)

        pltpu.emit_pipeline(
            body,
            grid=(_N // _W,),
            in_specs=[pl.BlockSpec((_W,), index_map=lambda i: (i,))],
            out_specs=[pl.BlockSpec((_W, _D), index_map=lambda i: (i, 0))],
            core_axis_name=("c", "s"),
            dimension_semantics=(pltpu.PARALLEL,),
        )(idx_hbm, out_hbm)

    wide = gather_kernel(table, idx).reshape(_S, _H, 128)

    swapped = pl.pallas_call(
        _swap_body,
        grid=(_S // _SB,),
        in_specs=[pl.BlockSpec((_SB, _H, 128), lambda i: (i, 0, 0))],
        out_specs=pl.BlockSpec((_SB, _D, _B), lambda i: (i, 0, 0)),
        out_shape=jax.ShapeDtypeStruct((_S, _D, _B), emb_weight.dtype),
    )(wide)

    return swapped.transpose(2, 0, 1)


# speedup vs baseline: 1.2074x; 1.0738x over previous
import functools

import jax
import jax.numpy as jnp
from jax.experimental import pallas as pl
from jax.experimental.pallas import tpu as pltpu
from jax.experimental.pallas import tpu_sc as plsc

_B, _S, _D = 4096, 50, 64
_N = _B * _S
_V = 100001
_W = 400
_TT = 8192
_SB = 5
_SH = _S // 2
_NH = _SH * _B


def _prep_body(i_ref, o_ref):
    t = jnp.transpose(i_ref[...], (1, 0))
    o_ref[...] = jnp.concatenate([t, t], axis=1)


def _swap_body(i_ref, o_ref):
    o_ref[...] = jnp.transpose(i_ref[:, :, : _D], (0, 2, 1))


def _swap_body2(acc_ref, i_ref, o_ref):
    del acc_ref
    o_ref[...] = jnp.transpose(i_ref[:, :, : _D], (0, 2, 1))


def _make_gather():
    @functools.partial(
        pl.kernel,
        out_type=jax.ShapeDtypeStruct((_NH, 128), jnp.float32),
        mesh=plsc.VectorSubcoreMesh(core_axis_name="c", subcore_axis_name="s"),
    )
    def gather_kernel(table_hbm, idx_hbm, out_hbm):
        def body(idx_vmem, out_vmem):
            pltpu.sync_copy(table_hbm.at[idx_vmem], out_vmem)

        pltpu.emit_pipeline(
            body,
            grid=(_NH // _W,),
            in_specs=[pl.BlockSpec((_W,), index_map=lambda i: (i,))],
            out_specs=[pl.BlockSpec((_W, 128), index_map=lambda i: (i, 0))],
            core_axis_name=("c", "s"),
            dimension_semantics=(pltpu.PARALLEL,),
        )(idx_hbm, out_hbm)

    return gather_kernel


def kernel(x, emb_weight):
    idx = x.T.reshape(_N).astype(jnp.int32)

    table = pl.pallas_call(
        _prep_body,
        grid=(pl.cdiv(_V, _TT),),
        in_specs=[pl.BlockSpec((_D, _TT), lambda i: (0, i))],
        out_specs=pl.BlockSpec((_TT, 128), lambda i: (i, 0)),
        out_shape=jax.ShapeDtypeStruct((_V, 128), emb_weight.dtype),
    )(emb_weight.T)

    gather = _make_gather()
    wide1 = gather(table, idx[:_NH]).reshape(_SH, _B, 128)
    wide2 = gather(table, idx[_NH:]).reshape(_SH, _B, 128)

    half1 = pl.pallas_call(
        _swap_body,
        grid=(_SH // _SB,),
        in_specs=[pl.BlockSpec((_SB, _B, 128), lambda i: (i, 0, 0))],
        out_specs=pl.BlockSpec((_SB, _D, _B), lambda i: (i, 0, 0)),
        out_shape=jax.ShapeDtypeStruct((_S, _D, _B), emb_weight.dtype),
    )(wide1)

    swapped = pl.pallas_call(
        _swap_body2,
        grid=(_SH // _SB,),
        in_specs=[
            pl.BlockSpec(memory_space=pltpu.MemorySpace.HBM),
            pl.BlockSpec((_SB, _B, 128), lambda i: (i, 0, 0)),
        ],
        out_specs=pl.BlockSpec((_SB, _D, _B), lambda i: (i + _SH // _SB, 0, 0)),
        out_shape=jax.ShapeDtypeStruct((_S, _D, _B), emb_weight.dtype),
        input_output_aliases={0: 0},
    )(half1, wide2)

    return swapped.transpose(2, 0, 1)


# speedup vs baseline: 1.2323x; 1.0207x over previous
import functools

import jax
import jax.numpy as jnp
from jax.experimental import pallas as pl
from jax.experimental.pallas import tpu as pltpu
from jax.experimental.pallas import tpu_sc as plsc

_B, _S, _D = 4096, 50, 64
_N = _B * _S
_V = 100001
_W = 400
_TT = 8192
_SB = 5


def _prep_body(i_ref, o_ref):
    t = jnp.transpose(i_ref[...], (1, 0))
    o_ref[...] = jnp.concatenate([t, t], axis=1)


def _swap_body(i_ref, o_ref):
    o_ref[...] = jnp.transpose(i_ref[:, :, : _D], (0, 2, 1))


def kernel(x, emb_weight):
    idx = x.T.reshape(_N).astype(jnp.int32)

    table = pl.pallas_call(
        _prep_body,
        grid=(pl.cdiv(_V, _TT),),
        in_specs=[pl.BlockSpec((_D, _TT), lambda i: (0, i))],
        out_specs=pl.BlockSpec((_TT, 128), lambda i: (i, 0)),
        out_shape=jax.ShapeDtypeStruct((_V, 128), emb_weight.dtype),
    )(emb_weight.T)

    @functools.partial(
        pl.kernel,
        out_type=jax.ShapeDtypeStruct((_N, 128), emb_weight.dtype),
        mesh=plsc.VectorSubcoreMesh(core_axis_name="c", subcore_axis_name="s"),
    )
    def gather_kernel(table_hbm, idx_hbm, out_hbm):
        def body(idx_vmem, out_vmem):
            pltpu.sync_copy(table_hbm.at[idx_vmem], out_vmem)

        pltpu.emit_pipeline(
            body,
            grid=(_N // _W,),
            in_specs=[pl.BlockSpec((_W,), index_map=lambda i: (i,))],
            out_specs=[pl.BlockSpec((_W, 128), index_map=lambda i: (i, 0))],
            core_axis_name=("c", "s"),
            dimension_semantics=(pltpu.PARALLEL,),
        )(idx_hbm, out_hbm)

    wide = gather_kernel(table, idx).reshape(_S, _B, 128)

    swapped = pl.pallas_call(
        _swap_body,
        grid=(_S // _SB,),
        in_specs=[pl.BlockSpec((_SB, _B, 128), lambda i: (i, 0, 0))],
        out_specs=pl.BlockSpec((_SB, _D, _B), lambda i: (i, 0, 0)),
        out_shape=jax.ShapeDtypeStruct((_S, _D, _B), emb_weight.dtype),
    )(wide)

    return swapped.transpose(2, 0, 1)
